# baseline (speedup 1.0000x reference)
import jax
import jax.numpy as jnp
from jax.experimental import pallas as pl

N = 2048
D = 768
H = 768
E = 8
K = 2
TN = 256


def _moe_body(logits_ref, x_ref, W1_ref, b1_ref, W2_ref, b2_ref,
              out_ref, probs_ref, mask_ref):
    l = logits_ref[...]
    m = jnp.max(l, axis=-1, keepdims=True)
    ex = jnp.exp(l - m)
    probs = ex / jnp.sum(ex, axis=-1, keepdims=True)
    probs_ref[...] = probs

    iota = jax.lax.broadcasted_iota(jnp.int32, (TN, E), 1)
    m1 = jnp.max(probs, axis=-1, keepdims=True)
    i1 = jnp.min(jnp.where(probs == m1, iota, E), axis=-1, keepdims=True)
    oh1 = iota == i1
    probs2 = jnp.where(oh1, -1.0, probs)
    m2 = jnp.max(probs2, axis=-1, keepdims=True)
    i2 = jnp.min(jnp.where(probs2 == m2, iota, E), axis=-1, keepdims=True)
    oh2 = iota == i2
    denom = m1 + m2 + 1e-8
    mask = jnp.where(oh1, m1 / denom, 0.0) + jnp.where(oh2, m2 / denom, 0.0)
    mask_ref[...] = mask

    xb = x_ref[...].astype(jnp.bfloat16)
    acc = jnp.zeros((TN, H), jnp.float32)
    for e in range(E):
        h = jnp.dot(xb, W1_ref[e], preferred_element_type=jnp.float32)
        h = jnp.maximum(h + b1_ref[e], 0.0).astype(jnp.bfloat16)
        y = jnp.dot(h, W2_ref[e], preferred_element_type=jnp.float32)
        acc = acc + (y + b2_ref[e]) * mask[:, e:e + 1]
    out_ref[...] = acc


def kernel(x, Wg, bg, W1, b1, W2, b2):
    gate_logits = x @ Wg + bg
    W1b = W1.astype(jnp.bfloat16)
    W2b = W2.astype(jnp.bfloat16)
    b1r = b1[:, None, :]
    b2r = b2[:, None, :]

    grid = (N // TN,)
    out, probs, mask = pl.pallas_call(
        _moe_body,
        grid=grid,
        in_specs=[
            pl.BlockSpec((TN, E), lambda i: (i, 0)),
            pl.BlockSpec((TN, D), lambda i: (i, 0)),
            pl.BlockSpec((E, D, H), lambda i: (0, 0, 0)),
            pl.BlockSpec((E, 1, H), lambda i: (0, 0, 0)),
            pl.BlockSpec((E, H, H), lambda i: (0, 0, 0)),
            pl.BlockSpec((E, 1, H), lambda i: (0, 0, 0)),
        ],
        out_specs=[
            pl.BlockSpec((TN, H), lambda i: (i, 0)),
            pl.BlockSpec((TN, E), lambda i: (i, 0)),
            pl.BlockSpec((TN, E), lambda i: (i, 0)),
        ],
        out_shape=[
            jax.ShapeDtypeStruct((N, H), jnp.float32),
            jax.ShapeDtypeStruct((N, E), jnp.float32),
            jax.ShapeDtypeStruct((N, E), jnp.float32),
        ],
    )(gate_logits, x, W1b, b1r, W2b, b2r)
    return (out, probs, mask)
